# baseline (speedup 1.0000x reference)
import functools

import jax
import jax.numpy as jnp
from jax import lax
from jax.experimental import pallas as pl
from jax.experimental.pallas import tpu as pltpu
from jax.experimental.pallas import tpu_sc as plsc

DIM = 32
L = 16

_info = plsc.get_sparse_core_info()
NC, NS = _info.num_cores, _info.num_subcores
NW = NC * NS

CH = 128
K = 8
GROUP = CH * K


def _gather_kernel(n_rows_2d, n_w, n_groups):
    mesh = plsc.VectorSubcoreMesh(core_axis_name="c", subcore_axis_name="s")

    @functools.partial(
        pl.kernel,
        mesh=mesh,
        out_type=jax.ShapeDtypeStruct((n_rows_2d, CH, DIM), jnp.float32),
        scratch_types=[
            pltpu.VMEM((K, CH), jnp.int32),
            pltpu.VMEM((K, CH, DIM), jnp.float32),
            pltpu.VMEM((L,), jnp.int32),
            pltpu.SemaphoreType.DMA,
        ],
        compiler_params=pltpu.CompilerParams(use_tc_tiling_on_sc=False),
    )
    def body(ids_hbm, off_hbm, table_hbm, out_hbm, idx_v, rows_v, off_v, sem):
        wid = lax.axis_index("s") * NC + lax.axis_index("c")
        row_base = wid * (n_w // CH)
        pltpu.sync_copy(off_hbm, off_v)
        off = off_v[...]

        def group(g, carry):
            gb = row_base + g * K
            pltpu.sync_copy(ids_hbm.at[pl.ds(gb, K)], idx_v)
            for j in range(K):
                for i in range(CH // L):
                    s = pl.ds(i * L, L)
                    idx_v[j, s] = idx_v[j, s] + off
            copies = [
                pltpu.async_copy(table_hbm.at[idx_v.at[j]], rows_v.at[j], sem)
                for j in range(K)
            ]
            for c in copies:
                c.wait()
            pltpu.sync_copy(rows_v, out_hbm.at[pl.ds(gb, K)])
            return carry

        lax.fori_loop(0, n_groups, group, 0)

    return body


def kernel(input_ids, offsets, W):
    B, S, H = input_ids.shape
    N = B * S * H
    n_w = N // NW
    n_groups = n_w // GROUP
    ids2d = input_ids.reshape(N // CH, CH)
    offs16 = jnp.tile(offsets, L // H)
    out = _gather_kernel(N // CH, n_w, n_groups)(ids2d, offs16, W)
    return out.reshape(B, S, H, DIM)


# speedup vs baseline: 1.0048x; 1.0048x over previous
import functools

import jax
import jax.numpy as jnp
from jax import lax
from jax.experimental import pallas as pl
from jax.experimental.pallas import tpu as pltpu
from jax.experimental.pallas import tpu_sc as plsc

DIM = 32
L = 16

_info = plsc.get_sparse_core_info()
NC, NS = _info.num_cores, _info.num_subcores
NW = NC * NS

CH = 128
DEPTH = 24
LEAD = 6


def _gather_kernel(n_rows_2d, n_streams):
    mesh = plsc.VectorSubcoreMesh(core_axis_name="c", subcore_axis_name="s")

    @functools.partial(
        pl.kernel,
        mesh=mesh,
        out_type=jax.ShapeDtypeStruct((n_rows_2d, CH, DIM), jnp.float32),
        scratch_types=[
            pltpu.VMEM((n_streams, CH), jnp.int32),
            pltpu.VMEM((DEPTH, CH, DIM), jnp.float32),
            pltpu.VMEM((L,), jnp.int32),
            pltpu.SemaphoreType.DMA,
            pltpu.SemaphoreType.DMA,
        ],
        compiler_params=pltpu.CompilerParams(use_tc_tiling_on_sc=False),
    )
    def body(ids_hbm, off_hbm, table_hbm, out_hbm, idx_v, rows_v, off_v,
             g_sem, w_sem):
        wid = lax.axis_index("s") * NC + lax.axis_index("c")
        row_base = wid * n_streams
        pltpu.sync_copy(off_hbm, off_v)
        pltpu.sync_copy(ids_hbm.at[pl.ds(row_base, n_streams)], idx_v)
        off = off_v[...]

        def shift(j, carry):
            for i in range(CH // L):
                s = pl.ds(i * L, L)
                idx_v[j, s] = idx_v[j, s] + off
            return carry

        lax.fori_loop(0, n_streams, shift, 0)

        g_copies = [None] * n_streams
        w_copies = [None] * n_streams

        def fire_write(j):
            g_copies[j].wait()
            w_copies[j] = pltpu.async_copy(
                rows_v.at[j % DEPTH], out_hbm.at[row_base + j], w_sem)

        for j in range(n_streams):
            if j >= DEPTH:
                w_copies[j - DEPTH].wait()
            g_copies[j] = pltpu.async_copy(
                table_hbm.at[idx_v.at[j]], rows_v.at[j % DEPTH], g_sem)
            if j >= LEAD:
                fire_write(j - LEAD)
        for j in range(n_streams - LEAD, n_streams):
            fire_write(j)
        for j in range(max(n_streams - DEPTH, 0), n_streams):
            w_copies[j].wait()

    return body


def kernel(input_ids, offsets, W):
    B, S, H = input_ids.shape
    N = B * S * H
    n_streams = N // NW // CH
    ids2d = input_ids.reshape(N // CH, CH)
    offs16 = jnp.tile(offsets, L // H)
    out = _gather_kernel(N // CH, n_streams)(ids2d, offs16, W)
    return out.reshape(B, S, H, DIM)
